# baseline (speedup 1.0000x reference)
import functools

import jax
import jax.numpy as jnp
from jax import lax
from jax.experimental import pallas as pl
from jax.experimental.pallas import tpu as pltpu
from jax.experimental.pallas import tpu_sc as plsc

LANES = 16
CHUNK = 8
NBUF = 4
AHEAD = 2
SEQ_BLK = 512


@functools.cache
def _build_sc_gather(n_rows: int, d: int):
    info = plsc.get_sparse_core_info()
    nw = info.num_cores * info.num_subcores
    rows_per_w = n_rows // nw
    n_chunks = rows_per_w // CHUNK
    n_ctiles = d // 128
    assert rows_per_w % CHUNK == 0 and d % 128 == 0
    assert n_chunks % NBUF == 0

    mesh = plsc.VectorSubcoreMesh(core_axis_name="c", subcore_axis_name="s")

    @functools.partial(
        pl.kernel,
        mesh=mesh,
        out_type=jax.ShapeDtypeStruct(
            (n_rows // CHUNK, n_ctiles, CHUNK, 128), jnp.float32),
        scratch_types=[
            pltpu.VMEM((2 * rows_per_w,), jnp.int32),
            pltpu.VMEM((NBUF, 2 * CHUNK, d), jnp.float32),
            pltpu.VMEM((NBUF, 1, n_ctiles, CHUNK, 128), jnp.float32),
        ]
        + [pltpu.SemaphoreType.DMA] * (2 * NBUF),
    )
    def sc_kernel(iab_hbm, pe_hbm, g_hbm, iab_v, rab_v, o_v, *sems):
        sem_in = sems[:NBUF]
        sem_out = sems[NBUF:]
        wid = lax.axis_index("s") * info.num_cores + lax.axis_index("c")
        base_w = wid * rows_per_w
        rtile_w = wid * n_chunks

        pltpu.sync_copy(iab_hbm.at[pl.ds(2 * base_w, 2 * rows_per_w)], iab_v)

        def issue_in(ci, b):
            off = 2 * ci * CHUNK
            pltpu.async_copy(
                pe_hbm.at[iab_v.at[pl.ds(off, 2 * CHUNK)]], rab_v.at[b],
                sem_in[b])

        for p in range(AHEAD):
            issue_in(p, p)

        def super_body(i, carry):
            for b in range(NBUF):
                ci = i * NBUF + b
                bn = (b + AHEAD) % NBUF

                @pl.when(ci >= NBUF - AHEAD)
                def _():
                    pltpu.make_async_copy(
                        o_v.at[bn], g_hbm.at[pl.ds(0, 1)],
                        sem_out[bn]).wait()

                @pl.when(ci < n_chunks - AHEAD)
                def _():
                    issue_in(ci + AHEAD, bn)

                pltpu.make_async_copy(
                    pe_hbm.at[pl.ds(0, 2 * CHUNK)], rab_v.at[b],
                    sem_in[b]).wait()

                def row_body(r, rcarry):
                    for j in range(d // LANES):
                        c, l = j // 8, (j % 8) * LANES
                        s = j * LANES
                        o_v[b, 0, c, r, pl.ds(l, LANES)] = (
                            rab_v[b, 2 * r, pl.ds(s, LANES)]
                            + rab_v[b, 2 * r + 1, pl.ds(s, LANES)]) * 0.001
                    return rcarry

                lax.fori_loop(0, CHUNK, row_body, 0)

                pltpu.async_copy(
                    o_v.at[b], g_hbm.at[pl.ds(rtile_w + ci, 1)], sem_out[b])
            return carry

        lax.fori_loop(0, n_chunks // NBUF, super_body, 0)

        for ci in range(n_chunks - AHEAD, n_chunks):
            b = ci % NBUF
            pltpu.make_async_copy(
                o_v.at[b], g_hbm.at[pl.ds(0, 1)], sem_out[b]).wait()

    return sc_kernel


def _tc_add_body(x_ref, g_ref, o_ref):
    gb = g_ref[...]
    gt = jnp.transpose(gb, (0, 2, 1, 3))
    nrows = gb.shape[0] * 8
    s, bt, d = x_ref.shape
    g3 = gt.reshape(nrows, d).reshape(s, bt, d)
    o_ref[...] = x_ref[...] + g3


@functools.cache
def _build_tc_add(seq: int, batch: int, d: int):
    n_rblk = SEQ_BLK * batch // 8
    return pl.pallas_call(
        _tc_add_body,
        grid=(seq // SEQ_BLK,),
        in_specs=[
            pl.BlockSpec((SEQ_BLK, batch, d), lambda i: (i, 0, 0)),
            pl.BlockSpec((n_rblk, d // 128, 8, 128), lambda i: (i, 0, 0, 0)),
        ],
        out_specs=pl.BlockSpec((SEQ_BLK, batch, d), lambda i: (i, 0, 0)),
        out_shape=jax.ShapeDtypeStruct((seq, batch, d), jnp.float32),
    )


def kernel(x, sec_pos_label, in_sec_pos_label, pe):
    seq, batch, d = x.shape
    n_rows = seq * batch
    ia = sec_pos_label.reshape(n_rows).astype(jnp.int32)
    ib = in_sec_pos_label.reshape(n_rows).astype(jnp.int32)
    iab = jnp.stack([ia, ib], axis=1).reshape(2 * n_rows)
    pe2 = pe.reshape(pe.shape[0], d)
    g = _build_sc_gather(n_rows, d)(iab, pe2)
    return _build_tc_add(seq, batch, d)(x, g)


# speedup vs baseline: 1.8985x; 1.8985x over previous
import functools

import jax
import jax.numpy as jnp
from jax import lax
from jax.experimental import pallas as pl
from jax.experimental.pallas import tpu as pltpu
from jax.experimental.pallas import tpu_sc as plsc

LANES = 16
CHUNK = 8
NBUF = 4
AHEAD = 2
SEQ_BLK = 512


@functools.cache
def _build_sc_gather(n_rows: int, d: int):
    info = plsc.get_sparse_core_info()
    nw = info.num_cores * info.num_subcores
    rows_per_w = n_rows // nw
    n_chunks = rows_per_w // CHUNK
    n_ctiles = d // 128
    assert rows_per_w % CHUNK == 0 and d % 128 == 0
    assert n_chunks % NBUF == 0

    mesh = plsc.VectorSubcoreMesh(core_axis_name="c", subcore_axis_name="s")

    @functools.partial(
        pl.kernel,
        mesh=mesh,
        out_type=jax.ShapeDtypeStruct(
            (n_rows // CHUNK, n_ctiles, CHUNK, 128), jnp.float32),
        scratch_types=[
            pltpu.VMEM((rows_per_w,), jnp.int32),
            pltpu.VMEM((rows_per_w,), jnp.int32),
            pltpu.VMEM((NBUF, CHUNK, d), jnp.float32),
            pltpu.VMEM((NBUF, CHUNK, d), jnp.float32),
            pltpu.VMEM((NBUF, 1, n_ctiles, CHUNK, 128), jnp.float32),
        ]
        + [pltpu.SemaphoreType.DMA] * (2 * NBUF),
    )
    def sc_kernel(ia_hbm, ib_hbm, pe_hbm, g_hbm,
                  ia_v, ib_v, ra_v, rb_v, o_v, *sems):
        sem_in = sems[:NBUF]
        sem_out = sems[NBUF:]
        wid = lax.axis_index("s") * info.num_cores + lax.axis_index("c")
        base_w = wid * rows_per_w
        rtile_w = wid * n_chunks

        pltpu.sync_copy(ia_hbm.at[pl.ds(base_w, rows_per_w)], ia_v)
        pltpu.sync_copy(ib_hbm.at[pl.ds(base_w, rows_per_w)], ib_v)

        def issue_in(ci, b):
            off = ci * CHUNK
            pltpu.async_copy(
                pe_hbm.at[ia_v.at[pl.ds(off, CHUNK)]], ra_v.at[b], sem_in[b])
            pltpu.async_copy(
                pe_hbm.at[ib_v.at[pl.ds(off, CHUNK)]], rb_v.at[b], sem_in[b])

        for p in range(AHEAD):
            issue_in(p, p)

        def super_body(i, carry):
            for b in range(NBUF):
                ci = i * NBUF + b
                bn = (b + AHEAD) % NBUF

                @pl.when(ci >= NBUF - AHEAD)
                def _():
                    pltpu.make_async_copy(
                        o_v.at[bn], g_hbm.at[pl.ds(0, 1)],
                        sem_out[bn]).wait()

                @pl.when(ci < n_chunks - AHEAD)
                def _():
                    issue_in(ci + AHEAD, bn)

                pltpu.make_async_copy(
                    pe_hbm.at[pl.ds(0, CHUNK)], ra_v.at[b],
                    sem_in[b]).wait()
                pltpu.make_async_copy(
                    pe_hbm.at[pl.ds(0, CHUNK)], rb_v.at[b],
                    sem_in[b]).wait()

                def row_body(r, rcarry):
                    for j in range(d // LANES):
                        c, l = j // 8, (j % 8) * LANES
                        s = j * LANES
                        o_v[b, 0, c, r, pl.ds(l, LANES)] = (
                            ra_v[b, r, pl.ds(s, LANES)]
                            + rb_v[b, r, pl.ds(s, LANES)]) * 0.001
                    return rcarry

                lax.fori_loop(0, CHUNK, row_body, 0)

                pltpu.async_copy(
                    o_v.at[b], g_hbm.at[pl.ds(rtile_w + ci, 1)], sem_out[b])
            return carry

        lax.fori_loop(0, n_chunks // NBUF, super_body, 0)

        for ci in range(n_chunks - AHEAD, n_chunks):
            b = ci % NBUF
            pltpu.make_async_copy(
                o_v.at[b], g_hbm.at[pl.ds(0, 1)], sem_out[b]).wait()

    return sc_kernel


def _tc_add_body(x_ref, g_ref, o_ref):
    gb = g_ref[...]
    gt = jnp.transpose(gb, (0, 2, 1, 3))
    nrows = gb.shape[0] * 8
    s, bt, d = x_ref.shape
    g3 = gt.reshape(nrows, d).reshape(s, bt, d)
    o_ref[...] = x_ref[...] + g3


@functools.cache
def _build_tc_add(seq: int, batch: int, d: int):
    n_rblk = SEQ_BLK * batch // 8
    return pl.pallas_call(
        _tc_add_body,
        grid=(seq // SEQ_BLK,),
        in_specs=[
            pl.BlockSpec((SEQ_BLK, batch, d), lambda i: (i, 0, 0)),
            pl.BlockSpec((n_rblk, d // 128, 8, 128), lambda i: (i, 0, 0, 0)),
        ],
        out_specs=pl.BlockSpec((SEQ_BLK, batch, d), lambda i: (i, 0, 0)),
        out_shape=jax.ShapeDtypeStruct((seq, batch, d), jnp.float32),
    )


def kernel(x, sec_pos_label, in_sec_pos_label, pe):
    seq, batch, d = x.shape
    n_rows = seq * batch
    ia = sec_pos_label.reshape(n_rows).astype(jnp.int32)
    ib = in_sec_pos_label.reshape(n_rows).astype(jnp.int32)
    pe2 = pe.reshape(pe.shape[0], d)
    g = _build_sc_gather(n_rows, d)(ia, ib, pe2)
    return _build_tc_add(seq, batch, d)(x, g)
